# baseline (speedup 1.0000x reference)
import functools

import jax
import jax.numpy as jnp
from jax import lax
from jax.experimental import pallas as pl
from jax.experimental.pallas import tpu as pltpu
from jax.experimental.pallas import tpu_sc as plsc

BATCH = 4096
SEQ = 200
D = 64
NW = 32
BB = BATCH // NW
L = 16

_mesh = plsc.VectorSubcoreMesh(core_axis_name="c", subcore_axis_name="s")


@functools.partial(
    pl.kernel,
    mesh=_mesh,
    out_type=jax.ShapeDtypeStruct((SEQ, 8, NW, 8, 128), jnp.float32),
    scratch_types=[
        pltpu.VMEM((8, 128), jnp.int32),
        pltpu.VMEM((128,), jnp.int32),
        pltpu.VMEM((128,), jnp.int32),
        pltpu.VMEM((128, 128), jnp.float32),
        pltpu.VMEM((64, 128), jnp.float32),
        pltpu.VMEM((8, D), jnp.float32),
        pltpu.SemaphoreType.DMA,
    ],
    compiler_params=pltpu.CompilerParams(
        use_tc_tiling_on_sc=True, needs_layout_passes=False),
)
def _emb_kernel(xt_hbm, pos_hbm, tab2_hbm, out_hbm,
                idxt_v, idx2_v, base_v, rows_v, slab_v, posb_v, gsem):
    wid = lax.axis_index("s") * 2 + lax.axis_index("c")
    iota = lax.iota(jnp.int32, L)

    def s8_body(s8, _):
        pltpu.sync_copy(xt_hbm.at[pl.ds(s8 * 8, 8), pl.ds(wid * 128, 128)],
                        idxt_v)
        pltpu.sync_copy(pos_hbm.at[0, pl.ds(s8 * 8, 8)], posb_v)

        def sj_body(sj, _):
            s = s8 * 8 + sj
            for g in range(8):
                sl = pl.ds(g * L, L)
                tv = idxt_v[sj, sl]
                idx2_v[sl] = lax.shift_right_logical(tv, 1)
                base_v[sl] = (tv & 1) * 64
            pltpu.async_copy(tab2_hbm.at[idx2_v], rows_v, gsem).wait()

            def c_body(c, _):
                pvec = plsc.load_gather(
                    posb_v, [jnp.broadcast_to(sj, (L,)),
                             jnp.broadcast_to(c, (L,))])
                for g in range(8):
                    sl = pl.ds(g * L, L)
                    rg = iota + g * L
                    cg = base_v[sl] + c
                    val = plsc.load_gather(rows_v, [rg, cg]) + pvec
                    slab_v[c, sl] = val
                return 0

            lax.fori_loop(0, D, c_body, 0)
            for ci in range(8):
                pltpu.sync_copy(slab_v.at[pl.ds(ci * 8, 8)],
                                out_hbm.at[s, ci, wid])
            return 0

        lax.fori_loop(0, 8, sj_body, 0)
        return 0

    lax.fori_loop(0, SEQ // 8, s8_body, 0)


def kernel(x, token_emb_table, pos_emb):
    xt = jnp.transpose(x)
    tab2 = token_emb_table.reshape(500000, 128)
    out5 = _emb_kernel(xt, pos_emb, tab2)
    return out5.transpose(2, 4, 0, 1, 3).reshape(BATCH, SEQ, D)


# speedup vs baseline: 1.1359x; 1.1359x over previous
import functools

import jax
import jax.numpy as jnp
from jax import lax
from jax.experimental import pallas as pl
from jax.experimental.pallas import tpu as pltpu
from jax.experimental.pallas import tpu_sc as plsc

BATCH = 4096
SEQ = 200
D = 64
NW = 32
BB = BATCH // NW
L = 16

_mesh = plsc.VectorSubcoreMesh(core_axis_name="c", subcore_axis_name="s")


@functools.partial(
    pl.kernel,
    mesh=_mesh,
    out_type=jax.ShapeDtypeStruct((SEQ, 8, NW, 8, 128), jnp.float32),
    scratch_types=[
        pltpu.VMEM((8, 128), jnp.int32),
        [pltpu.VMEM((128,), jnp.int32)] * 2,
        [pltpu.VMEM((128,), jnp.int32)] * 2,
        [pltpu.VMEM((128, 128), jnp.float32)] * 2,
        [pltpu.VMEM((64, 128), jnp.float32)] * 2,
        pltpu.VMEM((8, D), jnp.float32),
        [pltpu.SemaphoreType.DMA] * 2,
        [pltpu.SemaphoreType.DMA] * 2,
    ],
    compiler_params=pltpu.CompilerParams(
        use_tc_tiling_on_sc=True, needs_layout_passes=False),
)
def _emb_kernel(xt_hbm, pos_hbm, tab2_hbm, out_hbm,
                idxt_v, idx2_v, base_v, rows_v, slab_v, posb_v, gsems, ssems):
    wid = lax.axis_index("s") * 2 + lax.axis_index("c")
    iota = lax.iota(jnp.int32, L)

    def build_idx(sj, b):
        for g in range(8):
            sl = pl.ds(g * L, L)
            tv = idxt_v[sj, sl]
            idx2_v[b][sl] = lax.shift_right_logical(tv, 1)
            base_v[b][sl] = (tv & 1) * 64

    def s8_body(s8, _):
        pltpu.sync_copy(xt_hbm.at[pl.ds(s8 * 8, 8), pl.ds(wid * 128, 128)],
                        idxt_v)
        pltpu.sync_copy(pos_hbm.at[0, pl.ds(s8 * 8, 8)], posb_v)

        build_idx(0, 0)
        pltpu.async_copy(tab2_hbm.at[idx2_v[0]], rows_v[0], gsems[0])
        for sj in range(8):
            b = sj % 2
            if sj < 7:
                nb = 1 - b
                build_idx(sj + 1, nb)
                pltpu.async_copy(
                    tab2_hbm.at[idx2_v[nb]], rows_v[nb], gsems[nb])
            pltpu.make_async_copy(
                tab2_hbm.at[idx2_v[b]], rows_v[b], gsems[b]).wait()
            if sj >= 2:
                for ci in range(8):
                    pltpu.make_async_copy(
                        slab_v[b].at[pl.ds(ci * 8, 8)],
                        out_hbm.at[s8 * 8 + sj - 2, ci, wid],
                        ssems[b]).wait()
            sjv = jnp.int32(sj)

            def c_body(c4, _):
                for u in range(4):
                    c = c4 * 4 + u
                    pvec = plsc.load_gather(
                        posb_v, [jnp.broadcast_to(sjv, (L,)),
                                 jnp.broadcast_to(c, (L,))])
                    for g in range(8):
                        sl = pl.ds(g * L, L)
                        rg = iota + g * L
                        cg = base_v[b][sl] + c
                        val = plsc.load_gather(rows_v[b], [rg, cg]) + pvec
                        slab_v[b][c, sl] = val
                return 0

            lax.fori_loop(0, D // 4, c_body, 0)
            for ci in range(8):
                pltpu.async_copy(slab_v[b].at[pl.ds(ci * 8, 8)],
                                 out_hbm.at[s8 * 8 + sj, ci, wid], ssems[b])
        for sj in (6, 7):
            b = sj % 2
            for ci in range(8):
                pltpu.make_async_copy(
                    slab_v[b].at[pl.ds(ci * 8, 8)],
                    out_hbm.at[s8 * 8 + sj, ci, wid], ssems[b]).wait()
        return 0

    lax.fori_loop(0, SEQ // 8, s8_body, 0)


def kernel(x, token_emb_table, pos_emb):
    xt = jnp.transpose(x)
    tab2 = token_emb_table.reshape(500000, 128)
    out5 = _emb_kernel(xt, pos_emb, tab2)
    return out5.transpose(2, 4, 0, 1, 3).reshape(BATCH, SEQ, D)
